# baseline (device time: 12957 ns/iter reference)
import jax
import jax.numpy as jnp
from jax import lax
from jax.experimental import pallas as pl
from jax.experimental.pallas import tpu as pltpu

N_CHUNKS = 4


def kernel(x, pi):
    shard_shape = x.shape
    rows = shard_shape[1]
    rows_per = rows // N_CHUNKS

    def body(pi_hbm, x_hbm, out_ref, pi_smem, xv_ref, comm_ref,
             pi_sem, stage_sems, send_sems, recv_sems):
        my_x = lax.axis_index("x")
        my_y = lax.axis_index("y")
        my_z = lax.axis_index("z")

        pi_dma = pltpu.make_async_copy(pi_hbm, pi_smem, pi_sem)
        pi_dma.start()
        stage_dmas = []
        for c in range(N_CHUNKS):
            sl = pl.ds(c * rows_per, rows_per)
            d = pltpu.make_async_copy(
                x_hbm.at[:, sl, :], xv_ref.at[:, sl, :], stage_sems.at[c]
            )
            d.start()
            stage_dmas.append(d)

        pi_dma.wait()
        peer_y = pi_smem[my_y]
        peer = (my_x, peer_y, my_z)

        barrier_sem = pltpu.get_barrier_semaphore()
        pl.semaphore_signal(
            barrier_sem, inc=1, device_id=peer,
            device_id_type=pl.DeviceIdType.MESH,
        )
        sl0 = pl.ds(0, rows_per)
        stage_dmas[0].wait()
        comm_ref[0, sl0, :] = xv_ref[0, sl0, :].astype(jnp.bfloat16)
        pl.semaphore_wait(barrier_sem, 1)

        rdmas = []
        for c in range(N_CHUNKS):
            sl = pl.ds(c * rows_per, rows_per)
            rdma = pltpu.make_async_remote_copy(
                src_ref=comm_ref.at[:, sl, :],
                dst_ref=out_ref.at[:, sl, :],
                send_sem=send_sems.at[c],
                recv_sem=recv_sems.at[c],
                device_id=peer,
                device_id_type=pl.DeviceIdType.MESH,
            )
            rdma.start()
            rdmas.append(rdma)
            if c + 1 < N_CHUNKS:
                nxt = pl.ds((c + 1) * rows_per, rows_per)
                stage_dmas[c + 1].wait()
                comm_ref[0, nxt, :] = xv_ref[0, nxt, :].astype(jnp.bfloat16)
        for rdma in rdmas:
            rdma.wait()

    out_shape = jax.ShapeDtypeStruct(shard_shape, jnp.bfloat16)
    return pl.pallas_call(
        body,
        out_shape=out_shape,
        in_specs=[
            pl.BlockSpec(memory_space=pl.ANY),
            pl.BlockSpec(memory_space=pl.ANY),
        ],
        out_specs=pl.BlockSpec(memory_space=pl.ANY),
        scratch_shapes=[
            pltpu.SMEM((2,), jnp.int32),
            pltpu.VMEM(shard_shape, jnp.float32),
            pltpu.VMEM(shard_shape, jnp.bfloat16),
            pltpu.SemaphoreType.DMA,
            pltpu.SemaphoreType.DMA((N_CHUNKS,)),
            pltpu.SemaphoreType.DMA((N_CHUNKS,)),
            pltpu.SemaphoreType.DMA((N_CHUNKS,)),
        ],
        compiler_params=pltpu.CompilerParams(collective_id=0),
    )(pi, x)


# device time: 12840 ns/iter; 1.0091x vs baseline; 1.0091x over previous
import jax
import jax.numpy as jnp
from jax import lax
from jax.experimental import pallas as pl
from jax.experimental.pallas import tpu as pltpu

N_SUB = 8


def kernel(x, pi):
    shard_shape = x.shape
    rows = shard_shape[1]
    cols = shard_shape[2]
    half = rows // 2
    sub = half // N_SUB

    def body(pi_ref, x_ref, out_ref, comm_ref,
             ysend, yrecv, zsend, zrecv):
        my_x = lax.axis_index("x")
        my_y = lax.axis_index("y")
        my_z = lax.axis_index("z")
        peer = (my_x, pi_ref[my_y], my_z)
        znbr = (my_x, my_y, 1 - my_z)
        base = my_z * half

        barrier_sem = pltpu.get_barrier_semaphore()
        for tgt in (peer, znbr):
            pl.semaphore_signal(
                barrier_sem, inc=1, device_id=tgt,
                device_id_type=pl.DeviceIdType.MESH,
            )
        comm_ref[0, pl.ds(0, sub), :] = (
            x_ref[0, pl.ds(base, sub), :].astype(jnp.bfloat16)
        )
        pl.semaphore_wait(barrier_sem, 2)

        yrd = []
        for k in range(N_SUB):
            r = pltpu.make_async_remote_copy(
                src_ref=comm_ref.at[:, pl.ds(k * sub, sub), :],
                dst_ref=out_ref.at[:, pl.ds(base + k * sub, sub), :],
                send_sem=ysend.at[k],
                recv_sem=yrecv.at[k],
                device_id=peer,
                device_id_type=pl.DeviceIdType.MESH,
            )
            r.start()
            yrd.append(r)
            if k + 1 < N_SUB:
                comm_ref[0, pl.ds((k + 1) * sub, sub), :] = (
                    x_ref[0, pl.ds(base + (k + 1) * sub, sub), :]
                    .astype(jnp.bfloat16)
                )

        zrd = []
        for k in range(N_SUB):
            yrd[k].wait_recv()
            r = pltpu.make_async_remote_copy(
                src_ref=out_ref.at[:, pl.ds(base + k * sub, sub), :],
                dst_ref=out_ref.at[:, pl.ds(base + k * sub, sub), :],
                send_sem=zsend.at[k],
                recv_sem=zrecv.at[k],
                device_id=znbr,
                device_id_type=pl.DeviceIdType.MESH,
            )
            r.start()
            zrd.append(r)

        for r in zrd:
            r.wait_recv()
        for k in range(N_SUB):
            yrd[k].wait_send()
            zrd[k].wait_send()

    out_shape = jax.ShapeDtypeStruct(shard_shape, jnp.bfloat16)
    return pl.pallas_call(
        body,
        out_shape=out_shape,
        in_specs=[
            pl.BlockSpec(memory_space=pltpu.SMEM),
            pl.BlockSpec(memory_space=pltpu.VMEM),
        ],
        out_specs=pl.BlockSpec(memory_space=pltpu.VMEM),
        scratch_shapes=[
            pltpu.VMEM((1, half, cols), jnp.bfloat16),
            pltpu.SemaphoreType.DMA((N_SUB,)),
            pltpu.SemaphoreType.DMA((N_SUB,)),
            pltpu.SemaphoreType.DMA((N_SUB,)),
            pltpu.SemaphoreType.DMA((N_SUB,)),
        ],
        compiler_params=pltpu.CompilerParams(collective_id=0),
    )(pi, x)


# device time: 12627 ns/iter; 1.0261x vs baseline; 1.0169x over previous
import jax
import jax.numpy as jnp
from jax import lax
from jax.experimental import pallas as pl
from jax.experimental.pallas import tpu as pltpu

N_CHUNKS = 4


def kernel(x, pi):
    shard_shape = x.shape
    rows = shard_shape[1]
    rows_per = rows // N_CHUNKS

    def body(pi_ref, x_ref, out_ref, comm_ref, send_sems, recv_sems):
        my_x = lax.axis_index("x")
        my_y = lax.axis_index("y")
        my_z = lax.axis_index("z")
        peer_y = pi_ref[my_y]
        peer = (my_x, peer_y, my_z)

        barrier_sem = pltpu.get_barrier_semaphore()
        pl.semaphore_signal(
            barrier_sem, inc=1, device_id=peer,
            device_id_type=pl.DeviceIdType.MESH,
        )
        comm_ref[0, pl.ds(0, rows_per), :] = (
            x_ref[0, pl.ds(0, rows_per), :].astype(jnp.bfloat16)
        )
        pl.semaphore_wait(barrier_sem, 1)

        rdmas = []
        for c in range(N_CHUNKS):
            sl = pl.ds(c * rows_per, rows_per)
            rdma = pltpu.make_async_remote_copy(
                src_ref=comm_ref.at[:, sl, :],
                dst_ref=out_ref.at[:, sl, :],
                send_sem=send_sems.at[c],
                recv_sem=recv_sems.at[c],
                device_id=peer,
                device_id_type=pl.DeviceIdType.MESH,
            )
            rdma.start()
            rdmas.append(rdma)
            if c + 1 < N_CHUNKS:
                nxt = pl.ds((c + 1) * rows_per, rows_per)
                comm_ref[0, nxt, :] = x_ref[0, nxt, :].astype(jnp.bfloat16)
        for rdma in rdmas:
            rdma.wait()

    out_shape = jax.ShapeDtypeStruct(shard_shape, jnp.bfloat16)
    return pl.pallas_call(
        body,
        out_shape=out_shape,
        in_specs=[
            pl.BlockSpec(memory_space=pltpu.SMEM),
            pl.BlockSpec(memory_space=pltpu.VMEM),
        ],
        out_specs=pl.BlockSpec(memory_space=pltpu.VMEM),
        scratch_shapes=[
            pltpu.VMEM(shard_shape, jnp.bfloat16),
            pltpu.SemaphoreType.DMA((N_CHUNKS,)),
            pltpu.SemaphoreType.DMA((N_CHUNKS,)),
        ],
        compiler_params=pltpu.CompilerParams(collective_id=0),
    )(pi, x)
